# baseline (speedup 1.0000x reference)
import functools

import jax
import jax.numpy as jnp
from jax import lax
from jax.experimental import pallas as pl
from jax.experimental.pallas import tpu as pltpu
from jax.experimental.pallas import tpu_sc as plsc

B = 4096
D = 64

_info = plsc.get_sparse_core_info()
_NC, _NS = _info.num_cores, _info.num_subcores
_NW = _NC * _NS
_BPW = B // _NW


def _sc_gather_body(user_ids, item_ids, cat_ids,
                    user_table, item_table, cat_table,
                    user_out, item_out, cat_out,
                    uidx_v, iidx_v, cidx_v,
                    urows_v, irows_v, crows_v,
                    sem_u, sem_i, sem_c):
    wid = lax.axis_index("s") * _NC + lax.axis_index("c")
    base = wid * _BPW
    sl = pl.ds(base, _BPW)
    pltpu.sync_copy(user_ids.at[sl], uidx_v)
    pltpu.sync_copy(item_ids.at[sl], iidx_v)
    pltpu.sync_copy(cat_ids.at[sl], cidx_v)
    cu = pltpu.async_copy(user_table.at[uidx_v], urows_v, sem_u)
    ci = pltpu.async_copy(item_table.at[iidx_v], irows_v, sem_i)
    cc = pltpu.async_copy(cat_table.at[cidx_v], crows_v, sem_c)
    cu.wait()
    pltpu.sync_copy(urows_v, user_out.at[sl])
    ci.wait()
    pltpu.sync_copy(irows_v, item_out.at[sl])
    cc.wait()
    pltpu.sync_copy(crows_v, cat_out.at[sl])


@jax.jit
def _sc_gather(user_ids, item_ids, cat_ids, user_table, item_table, cat_table):
    mesh = plsc.VectorSubcoreMesh(core_axis_name="c", subcore_axis_name="s")
    f = pl.kernel(
        _sc_gather_body,
        out_type=(
            jax.ShapeDtypeStruct((B, D), jnp.float32),
            jax.ShapeDtypeStruct((B, D), jnp.float32),
            jax.ShapeDtypeStruct((B, D), jnp.float32),
        ),
        mesh=mesh,
        scratch_types=[
            pltpu.VMEM((_BPW,), jnp.int32),
            pltpu.VMEM((_BPW,), jnp.int32),
            pltpu.VMEM((_BPW,), jnp.int32),
            pltpu.VMEM((_BPW, D), jnp.float32),
            pltpu.VMEM((_BPW, D), jnp.float32),
            pltpu.VMEM((_BPW, D), jnp.float32),
            pltpu.SemaphoreType.DMA,
            pltpu.SemaphoreType.DMA,
            pltpu.SemaphoreType.DMA,
        ],
        compiler_params=pltpu.CompilerParams(use_tc_tiling_on_sc=False),
    )
    return f(user_ids, item_ids, cat_ids, user_table, item_table, cat_table)


def _tc_body(tags_ref, table_ref, item_ref, cat_ref, out_ref):
    acc = jnp.dot(tags_ref[...], table_ref[...],
                  preferred_element_type=jnp.float32)
    out_ref[...] = acc + item_ref[...] + cat_ref[...]


_BM = 512


@jax.jit
def _tc_matmul_add(attr_tags, tags_table, item_rows, cat_rows):
    k = attr_tags.shape[1]
    grid = (B // _BM,)
    return pl.pallas_call(
        _tc_body,
        grid=grid,
        in_specs=[
            pl.BlockSpec((_BM, k), lambda i: (i, 0)),
            pl.BlockSpec((k, D), lambda i: (0, 0)),
            pl.BlockSpec((_BM, D), lambda i: (i, 0)),
            pl.BlockSpec((_BM, D), lambda i: (i, 0)),
        ],
        out_specs=pl.BlockSpec((_BM, D), lambda i: (i, 0)),
        out_shape=jax.ShapeDtypeStruct((B, D), jnp.float32),
        compiler_params=pltpu.CompilerParams(
            dimension_semantics=("arbitrary",),
        ),
    )(attr_tags, tags_table, item_rows, cat_rows)


def kernel(user_ids, item_ids, attr_category, attr_tags,
           user_table, item_table, category_table, tags_table):
    user_emb, item_rows, cat_rows = _sc_gather(
        user_ids.astype(jnp.int32), item_ids.astype(jnp.int32),
        attr_category.astype(jnp.int32),
        user_table, item_table, category_table)
    item_total = _tc_matmul_add(attr_tags, tags_table, item_rows, cat_rows)
    return (user_emb, item_total)


# speedup vs baseline: 1.3669x; 1.3669x over previous
import jax
import jax.numpy as jnp
from jax import lax
from jax.experimental import pallas as pl
from jax.experimental.pallas import tpu as pltpu
from jax.experimental.pallas import tpu_sc as plsc

B = 4096
D = 64
L = 16

_info = plsc.get_sparse_core_info()
_NC, _NS = _info.num_cores, _info.num_subcores
_NW = _NC * _NS
_BPW = B // _NW
_CH = 16


def _sc_gather_body(user_ids, item_ids, cat_ids,
                    user_table, item_table, cat_table,
                    user_out, item_out, cat_out,
                    idv_u, idv_i, idv_c,
                    ob_u, ob_i, ob_c,
                    sem_u, sem_i, sem_c, sem_o):
    wid = lax.axis_index("s") * _NC + lax.axis_index("c")
    base = wid * _BPW
    sl = pl.ds(base, _BPW)
    head = pl.ds(0, _BPW)
    pltpu.sync_copy(user_ids.at[sl], idv_u.at[head])
    pltpu.sync_copy(item_ids.at[sl], idv_i.at[head])
    pltpu.sync_copy(cat_ids.at[sl], idv_c.at[head])
    def fire(rr, carry):
        id_u = idv_u[pl.ds(rr, L)][0]
        id_i = idv_i[pl.ds(rr, L)][0]
        id_c = idv_c[pl.ds(rr, L)][0]
        pltpu.async_copy(user_table.at[id_u], ob_u.at[rr], sem_u)
        pltpu.async_copy(item_table.at[id_i], ob_i.at[rr], sem_i)
        pltpu.async_copy(cat_table.at[id_c], ob_c.at[rr], sem_c)
        return carry

    def drain(rr, carry):
        pltpu.make_async_copy(user_table.at[0], ob_u.at[rr], sem_u).wait()
        pltpu.make_async_copy(item_table.at[0], ob_i.at[rr], sem_i).wait()
        pltpu.make_async_copy(cat_table.at[0], ob_c.at[rr], sem_c).wait()
        return carry

    lax.fori_loop(0, _BPW, fire, 0)
    lax.fori_loop(0, _BPW, drain, 0)
    cu = pltpu.async_copy(ob_u, user_out.at[sl], sem_o)
    ci = pltpu.async_copy(ob_i, item_out.at[sl], sem_o)
    cc = pltpu.async_copy(ob_c, cat_out.at[sl], sem_o)
    cu.wait()
    ci.wait()
    cc.wait()


@jax.jit
def _sc_gather(user_ids, item_ids, cat_ids, user_table, item_table, cat_table):
    mesh = plsc.VectorSubcoreMesh(core_axis_name="c", subcore_axis_name="s")
    f = pl.kernel(
        _sc_gather_body,
        out_type=(
            jax.ShapeDtypeStruct((B, D), jnp.float32),
            jax.ShapeDtypeStruct((B, D), jnp.float32),
            jax.ShapeDtypeStruct((B, D), jnp.float32),
        ),
        mesh=mesh,
        scratch_types=[
            pltpu.VMEM((_BPW + L,), jnp.int32),
            pltpu.VMEM((_BPW + L,), jnp.int32),
            pltpu.VMEM((_BPW + L,), jnp.int32),
            pltpu.VMEM((_BPW, D), jnp.float32),
            pltpu.VMEM((_BPW, D), jnp.float32),
            pltpu.VMEM((_BPW, D), jnp.float32),
            pltpu.SemaphoreType.DMA,
            pltpu.SemaphoreType.DMA,
            pltpu.SemaphoreType.DMA,
            pltpu.SemaphoreType.DMA,
        ],
        compiler_params=pltpu.CompilerParams(use_tc_tiling_on_sc=True),
    )
    return f(user_ids, item_ids, cat_ids, user_table, item_table, cat_table)


def _tc_body(tags_ref, table_ref, item_ref, cat_ref, out_ref):
    acc = jnp.dot(tags_ref[...], table_ref[...],
                  preferred_element_type=jnp.float32)
    out_ref[...] = acc + item_ref[...] + cat_ref[...]


_BM = 512


@jax.jit
def _tc_matmul_add(attr_tags, tags_table, item_rows, cat_rows):
    k = attr_tags.shape[1]
    return pl.pallas_call(
        _tc_body,
        grid=(B // _BM,),
        in_specs=[
            pl.BlockSpec((_BM, k), lambda i: (i, 0)),
            pl.BlockSpec((k, D), lambda i: (0, 0)),
            pl.BlockSpec((_BM, D), lambda i: (i, 0)),
            pl.BlockSpec((_BM, D), lambda i: (i, 0)),
        ],
        out_specs=pl.BlockSpec((_BM, D), lambda i: (i, 0)),
        out_shape=jax.ShapeDtypeStruct((B, D), jnp.float32),
        compiler_params=pltpu.CompilerParams(
            dimension_semantics=("arbitrary",),
        ),
    )(attr_tags, tags_table, item_rows, cat_rows)


def kernel(user_ids, item_ids, attr_category, attr_tags,
           user_table, item_table, category_table, tags_table):
    user_emb, item_rows, cat_rows = _sc_gather(
        user_ids.astype(jnp.int32), item_ids.astype(jnp.int32),
        attr_category.astype(jnp.int32),
        user_table, item_table, category_table)
    item_total = _tc_matmul_add(attr_tags, tags_table, item_rows, cat_rows)
    return (user_emb, item_total)
